# Initial kernel scaffold; baseline (speedup 1.0000x reference)
import jax
import jax.numpy as jnp
from jax.experimental import pallas as pl


def kernel(x, edge_index, edge_attr, W1, as1, ad1, We1, ae1, b1, g1, be1, W2, as2, ad2, We2, ae2, b2, g2, be2):
    raise NotImplementedError("write your pallas kernel here")


# speedup vs baseline: 1.0000x; 1.0000x over previous
import jax
import jax.numpy as jnp
from jax.experimental import pallas as pl

N = 50000
H1 = 4
C1 = 64
C2 = 64


def _add_self_loops(edge_index, edge_attr, num_nodes):
    dst = edge_index[1]
    sums = jnp.zeros((num_nodes, edge_attr.shape[1]), dtype=edge_attr.dtype).at[dst].add(edge_attr)
    cnt = jnp.zeros((num_nodes,), dtype=edge_attr.dtype).at[dst].add(1.0)
    mean = sums / jnp.clip(cnt, 1.0, None)[:, None]
    loop = jnp.arange(num_nodes, dtype=edge_index.dtype)
    ei = jnp.concatenate([edge_index, jnp.stack([loop, loop])], axis=1)
    ea = jnp.concatenate([edge_attr, mean], axis=0)
    return ei, ea


def _gat_conv(x, edge_index, edge_attr, W, att_src, att_dst, W_e, att_e, bias, heads, out_ch, concat, num_nodes):
    ei, ea = _add_self_loops(edge_index, edge_attr, num_nodes)
    src, dst = ei[0], ei[1]
    xh = (x @ W).reshape(num_nodes, heads, out_ch)
    a_src = jnp.sum(xh * att_src, axis=-1)
    a_dst = jnp.sum(xh * att_dst, axis=-1)
    eh = (ea @ W_e).reshape(-1, heads, out_ch)
    a_e = jnp.sum(eh * att_e, axis=-1)
    alpha = a_src[src] + a_dst[dst] + a_e
    alpha = jax.nn.leaky_relu(alpha, 0.2)
    amax = jax.ops.segment_max(alpha, dst, num_segments=num_nodes)
    amax = jnp.where(jnp.isfinite(amax), amax, 0.0)
    ex = jnp.exp(alpha - amax[dst])
    den = jax.ops.segment_sum(ex, dst, num_segments=num_nodes)
    att = ex / (den[dst] + 1e-16)
    msg = xh[src] * att[:, :, None]
    out = jax.ops.segment_sum(msg, dst, num_segments=num_nodes)
    if concat:
        out = out.reshape(num_nodes, heads * out_ch)
    else:
        out = out.mean(axis=1)
    return out + bias


def _ln_kernel(x_ref, g_ref, b_ref, o_ref):
    x = x_ref[...]
    mu = jnp.mean(x, axis=-1, keepdims=True)
    var = jnp.mean((x - mu) ** 2, axis=-1, keepdims=True)
    o_ref[...] = (x - mu) / jnp.sqrt(var + 1e-5) * g_ref[...] + b_ref[...]


def _layer_norm_pallas(x, g, b):
    n, d = x.shape
    blk = 1000
    return pl.pallas_call(
        _ln_kernel,
        out_shape=jax.ShapeDtypeStruct((n, d), x.dtype),
        grid=(n // blk,),
        in_specs=[
            pl.BlockSpec((blk, d), lambda i: (i, 0)),
            pl.BlockSpec((d,), lambda i: (0,)),
            pl.BlockSpec((d,), lambda i: (0,)),
        ],
        out_specs=pl.BlockSpec((blk, d), lambda i: (i, 0)),
    )(x, g, b)


def _layer_norm(x, g, b, eps=1e-5):
    mu = x.mean(-1, keepdims=True)
    var = x.var(-1, keepdims=True)
    return (x - mu) / jnp.sqrt(var + eps) * g + b


def kernel(x, edge_index, edge_attr, W1, as1, ad1, We1, ae1, b1, g1, be1, W2, as2, ad2, We2, ae2, b2, g2, be2):
    h = _gat_conv(x, edge_index, edge_attr, W1, as1, ad1, We1, ae1, b1, H1, C1, True, N)
    h = _layer_norm(h, g1, be1)
    h = jax.nn.relu(h)
    h = _gat_conv(h, edge_index, edge_attr, W2, as2, ad2, We2, ae2, b2, 1, C2, False, N)
    h = _layer_norm_pallas(h, g2, be2)
    return h


# speedup vs baseline: 19.1884x; 19.1880x over previous
import functools

import jax
import jax.numpy as jnp
from jax import lax
from jax.experimental import pallas as pl
from jax.experimental.pallas import tpu as pltpu
from jax.experimental.pallas import tpu_sc as plsc

N = 50000
E = 800000
H1 = 4
C1 = 64
C2 = 64

_NC = 2
_NS = 16
_NW = _NC * _NS


_ROWS = (E + 127) // 128
_RPW = 200
_RPAD = _RPW * _NW
_EPAD = _RPAD * 128
_NP = N + 48
_CHUNK = 8
_NCHUNK = _RPW // _CHUNK


def _easum_body(dst_hbm, ea8_hbm, zero_hbm, out_hbm, dstbuf, eabuf, acc):
    c = lax.axis_index("c")
    s = lax.axis_index("s")
    w = c * _NS + s
    zrows = _NP // _NS
    pltpu.sync_copy(zero_hbm.at[pl.ds(s * zrows, zrows)],
                    acc.at[pl.ds(s * zrows, zrows)])
    plsc.subcore_barrier()

    def chunk_body(i, _):
        row0 = w * _RPW + i * _CHUNK
        pltpu.sync_copy(dst_hbm.at[pl.ds(row0, _CHUNK)], dstbuf)
        pltpu.sync_copy(ea8_hbm.at[pl.ds(row0 * 128, _CHUNK * 128)], eabuf)
        for j in range(_CHUNK):
            pltpu.sync_copy(eabuf.at[pl.ds(j * 128, 128)],
                            acc.at[dstbuf.at[j]], add=True)
        return 0

    lax.fori_loop(0, _NCHUNK, chunk_body, 0)
    plsc.subcore_barrier()
    frows = _NP // _NS
    pltpu.sync_copy(acc.at[pl.ds(s * frows, frows)],
                    out_hbm.at[c, pl.ds(s * frows, frows)])


@jax.jit
def _easum_sc(dst_pad, ea8_pad, zero8):
    mesh = plsc.VectorSubcoreMesh(core_axis_name="c", subcore_axis_name="s")
    f = pl.kernel(
        _easum_body,
        out_type=jax.ShapeDtypeStruct((_NC, _NP, 8), jnp.float32),
        mesh=mesh,
        scratch_types=[
            pltpu.VMEM((_CHUNK, 128), jnp.int32),
            pltpu.VMEM((_CHUNK * 128, 8), jnp.float32),
            pltpu.VMEM_SHARED((_NP, 8), jnp.float32),
        ],
        compiler_params=pltpu.CompilerParams(use_tc_tiling_on_sc=False, needs_layout_passes=False),
    )
    return f(dst_pad, ea8_pad, zero8)


def _self_loop_mean(edge_index, edge_attr):
    dst = edge_index[1]
    dst_pad = jnp.concatenate(
        [dst, jnp.full((_EPAD - E,), N, dtype=jnp.int32)]).reshape(_RPAD, 128)
    ea8 = jnp.concatenate(
        [edge_attr, jnp.ones((E, 1), jnp.float32), jnp.zeros((E, 1), jnp.float32)],
        axis=1)
    ea8_pad = jnp.concatenate([ea8, jnp.zeros((_EPAD - E, 8), jnp.float32)], axis=0)
    zero8 = jnp.zeros((_NP, 8), jnp.float32)
    parts = _easum_sc(dst_pad, ea8_pad, zero8)
    tot = parts[0] + parts[1]
    sums = tot[:N, :6]
    cnt = tot[:N, 6]
    return sums / jnp.clip(cnt, 1.0, None)[:, None]


_ECHUNK = 2000
_EPT = E // _NS
_NCH = _EPT // _ECHUNK


def _make_edge_pass(heads, row, ch, rounds, nn):
    fpt = ch // _NS
    feat = row - 16

    def body(src_hbm, dst_hbm, ae_hbm, xh_hbm, adst_hbm, init_hbm, out_hbm,
             srcb, dstb, aeb, adstb, matchb, gixb, scixb, gbuf, exb, sem, acc):
        c = lax.axis_index("c")
        s = lax.axis_index("s")
        iota = lax.iota(jnp.int32, 16)
        ones = jnp.ones((16,), jnp.int32)
        tailmask = (iota < heads).astype(jnp.float32)
        eh = iota // heads
        hh = iota % heads
        epg = 16 // heads

        def round_body(r, _r):
            k = 2 * r + c
            lo = pl.multiple_of(k * ch, 128)
            pltpu.sync_copy(
                init_hbm.at[pl.ds(pl.multiple_of(lo + s * fpt, 8), fpt)],
                acc.at[pl.ds(pl.multiple_of(s * fpt, 8), fpt)])
            pltpu.sync_copy(adst_hbm.at[pl.ds(pl.multiple_of(lo * heads, 8),
                                              ch * heads)],
                            adstb.at[pl.ds(0, ch * heads)])
            plsc.subcore_barrier()

            def chunk_body(ci, _c):
                base = s * _EPT + ci * _ECHUNK
                pltpu.sync_copy(src_hbm.at[pl.ds(base, _ECHUNK)],
                                srcb.at[pl.ds(0, _ECHUNK)])
                pltpu.sync_copy(dst_hbm.at[pl.ds(base, _ECHUNK)],
                                dstb.at[pl.ds(0, _ECHUNK)])
                pltpu.sync_copy(ae_hbm.at[pl.ds(base * heads, _ECHUNK * heads)],
                                aeb.at[pl.ds(0, _ECHUNK * heads)])
                dstb[pl.ds(_ECHUNK, 16)] = jnp.full((16,), ch, jnp.int32) + lo
                srcb[pl.ds(_ECHUNK, 16)] = jnp.zeros((16,), jnp.int32)
                aeb[pl.ds(_ECHUNK * heads, 16)] = jnp.zeros((16,), jnp.float32)

                def filt(v, cnt):
                    d16 = dstb[pl.ds(v * 16, 16)] - lo
                    m = (d16 >= 0) & (d16 < ch)
                    pos = cnt + plsc.cumsum(ones, mask=m) - 1
                    plsc.store_scatter(matchb, [pos], iota + v * 16, mask=m)
                    return cnt + jnp.sum(m.astype(jnp.int32))

                cnt = lax.fori_loop(0, _ECHUNK // 16, filt, 0)
                kpad = (cnt + 15) & ~15
                plsc.store_scatter(matchb, [cnt + iota],
                                   jnp.full((16,), _ECHUNK, jnp.int32),
                                   mask=iota < (kpad - cnt))

                def group(g, _g):
                    ids16 = plsc.load_gather(matchb, [g * 16 + iota])
                    src16 = plsc.load_gather(srcb, [ids16])
                    dloc16 = plsc.load_gather(dstb, [ids16]) - lo
                    gixb[...] = src16
                    scixb[...] = dloc16
                    pltpu.async_copy(xh_hbm.at[gixb], gbuf, sem).wait()
                    for q in range(heads):
                        eq = eh + q * epg
                        idq = plsc.load_gather(matchb, [g * 16 + eq])
                        asrc = plsc.load_gather(
                            gbuf, [eq, hh + feat])
                        dq = plsc.load_gather(dstb, [idq]) - lo
                        adst = plsc.load_gather(adstb, [dq * heads + hh])
                        ae = plsc.load_gather(aeb, [idq * heads + hh])
                        a = asrc + adst + ae
                        a = jnp.maximum(a, 0.0) + 0.2 * jnp.minimum(a, 0.0)
                        exb[pl.ds(q * 16, 16)] = jnp.exp(a)
                    for e in range(16):
                        exvec = exb[pl.ds(e * heads, 16)]
                        for j in range(feat // 16):
                            sc = exvec[(j * 16) // 64]
                            gbuf[e, pl.ds(j * 16, 16)] = (
                                gbuf[e, pl.ds(j * 16, 16)] * sc)
                        gbuf[e, pl.ds(feat, 16)] = exvec * tailmask
                    pltpu.sync_copy(gbuf, acc.at[scixb], add=True)
                    return _g

                lax.fori_loop(0, kpad // 16, group, 0)
                return _c

            lax.fori_loop(0, _NCH, chunk_body, 0)
            plsc.subcore_barrier()
            pltpu.sync_copy(
                acc.at[pl.ds(pl.multiple_of(s * fpt, 8), fpt)],
                out_hbm.at[pl.ds(pl.multiple_of(lo + s * fpt, 8), fpt)])
            plsc.subcore_barrier()
            return _r

        lax.fori_loop(0, rounds, round_body, 0)

    mesh = plsc.VectorSubcoreMesh(core_axis_name="c", subcore_axis_name="s")
    return pl.kernel(
        body,
        out_type=jax.ShapeDtypeStruct((nn, row), jnp.float32),
        mesh=mesh,
        scratch_types=[
            pltpu.VMEM((_ECHUNK + 16,), jnp.int32),
            pltpu.VMEM((_ECHUNK + 16,), jnp.int32),
            pltpu.VMEM((_ECHUNK * heads + 16,), jnp.float32),
            pltpu.VMEM(((ch + 8) * heads,), jnp.float32),
            pltpu.VMEM((_ECHUNK,), jnp.int32),
            pltpu.VMEM((16,), jnp.int32),
            pltpu.VMEM((16,), jnp.int32),
            pltpu.VMEM((16, row), jnp.float32),
            pltpu.VMEM((16 * heads + 16,), jnp.float32),
            pltpu.SemaphoreType.DMA,
            pltpu.VMEM_SHARED((ch + 8, row), jnp.float32),
        ],
        compiler_params=pltpu.CompilerParams(use_tc_tiling_on_sc=False, needs_layout_passes=False),
    )


_CH1, _R1, _NN1 = 5120, 5, 51200
_CH2, _R2, _NN2 = 12544, 2, 50176
_EP1 = _make_edge_pass(H1, H1 * C1 + 16, _CH1, _R1, _NN1)
_EP2 = _make_edge_pass(1, C2 + 16, _CH2, _R2, _NN2)


def _gat_conv_sc(ep, nn, x, src, dst, edge_attr, mean_ea, W, att_src,
                 att_dst, W_e, att_e, bias, heads, out_ch, num_nodes):
    feat = heads * out_ch
    row = feat + 16
    xh = (x @ W)
    xh3 = xh.reshape(num_nodes, heads, out_ch)
    a_src = jnp.sum(xh3 * att_src, axis=-1)
    a_dst = jnp.sum(xh3 * att_dst, axis=-1)
    Ve = (W_e.reshape(-1, heads, out_ch) * att_e).sum(-1)
    ae_flat = (edge_attr @ Ve).reshape(-1)
    a_e_loop = mean_ea @ Ve

    xh_ext = jnp.concatenate(
        [xh, a_src, jnp.zeros((num_nodes, 16 - heads), jnp.float32)], axis=1)
    adst_flat = jnp.concatenate(
        [a_dst.reshape(-1),
         jnp.zeros(((nn - num_nodes) * heads,), jnp.float32)])

    alpha_l = a_src + a_dst + a_e_loop
    alpha_l = jnp.maximum(alpha_l, 0.0) + 0.2 * jnp.minimum(alpha_l, 0.0)
    ex_l = jnp.exp(alpha_l)
    init_feat = (xh3 * ex_l[:, :, None]).reshape(num_nodes, feat)
    init = jnp.concatenate(
        [init_feat, ex_l, jnp.zeros((num_nodes, 16 - heads), jnp.float32)],
        axis=1)
    init = jnp.concatenate(
        [init, jnp.zeros((nn - num_nodes, row), jnp.float32)], axis=0)

    out_ext = ep(src, dst, ae_flat, xh_ext, adst_flat, init)
    num = out_ext[:num_nodes, :feat].reshape(num_nodes, heads, out_ch)
    den = out_ext[:num_nodes, feat:feat + heads]
    out = num / (den[:, :, None] + 1e-16)
    if heads > 1:
        out = out.reshape(num_nodes, feat)
    else:
        out = out[:, 0]
    return out + bias


def _layer_norm(x, g, b, eps=1e-5):
    mu = x.mean(-1, keepdims=True)
    var = x.var(-1, keepdims=True)
    return (x - mu) / jnp.sqrt(var + eps) * g + b


def kernel(x, edge_index, edge_attr, W1, as1, ad1, We1, ae1, b1, g1, be1,
           W2, as2, ad2, We2, ae2, b2, g2, be2):
    src, dst = edge_index[0], edge_index[1]
    mean_ea = _self_loop_mean(edge_index, edge_attr)
    h = _gat_conv_sc(_EP1, _NN1, x, src, dst, edge_attr, mean_ea, W1, as1,
                     ad1, We1, ae1, b1, H1, C1, N)
    h = _layer_norm(h, g1, be1)
    h = jax.nn.relu(h)
    h = _gat_conv_sc(_EP2, _NN2, h, src, dst, edge_attr, mean_ea, W2, as2,
                     ad2, We2, ae2, b2, 1, C2, N)
    h = _layer_norm(h, g2, be2)
    return h


# speedup vs baseline: 26.5477x; 1.3835x over previous
import functools

import jax
import jax.numpy as jnp
from jax import lax
from jax.experimental import pallas as pl
from jax.experimental.pallas import tpu as pltpu
from jax.experimental.pallas import tpu_sc as plsc

N = 50000
E = 800000
H1 = 4
C1 = 64
C2 = 64

_NC = 2
_NS = 16
_NW = _NC * _NS


_ROWS = (E + 127) // 128
_RPW = 200
_RPAD = _RPW * _NW
_EPAD = _RPAD * 128
_NP = N + 48
_CHUNK = 8
_NCHUNK = _RPW // _CHUNK


def _easum_body(dst_hbm, ea8_hbm, zero_hbm, out_hbm, dstbuf, eabuf, acc):
    c = lax.axis_index("c")
    s = lax.axis_index("s")
    w = c * _NS + s
    zrows = _NP // _NS
    pltpu.sync_copy(zero_hbm.at[pl.ds(s * zrows, zrows)],
                    acc.at[pl.ds(s * zrows, zrows)])
    plsc.subcore_barrier()

    def chunk_body(i, _):
        row0 = w * _RPW + i * _CHUNK
        pltpu.sync_copy(dst_hbm.at[pl.ds(row0, _CHUNK)], dstbuf)
        pltpu.sync_copy(ea8_hbm.at[pl.ds(row0 * 128, _CHUNK * 128)], eabuf)
        for j in range(_CHUNK):
            pltpu.sync_copy(eabuf.at[pl.ds(j * 128, 128)],
                            acc.at[dstbuf.at[j]], add=True)
        return 0

    lax.fori_loop(0, _NCHUNK, chunk_body, 0)
    plsc.subcore_barrier()
    frows = _NP // _NS
    pltpu.sync_copy(acc.at[pl.ds(s * frows, frows)],
                    out_hbm.at[c, pl.ds(s * frows, frows)])


@jax.jit
def _easum_sc(dst_pad, ea8_pad, zero8):
    mesh = plsc.VectorSubcoreMesh(core_axis_name="c", subcore_axis_name="s")
    f = pl.kernel(
        _easum_body,
        out_type=jax.ShapeDtypeStruct((_NC, _NP, 8), jnp.float32),
        mesh=mesh,
        scratch_types=[
            pltpu.VMEM((_CHUNK, 128), jnp.int32),
            pltpu.VMEM((_CHUNK * 128, 8), jnp.float32),
            pltpu.VMEM_SHARED((_NP, 8), jnp.float32),
        ],
        compiler_params=pltpu.CompilerParams(use_tc_tiling_on_sc=False, needs_layout_passes=False),
    )
    return f(dst_pad, ea8_pad, zero8)


def _self_loop_mean(edge_index, edge_attr):
    dst = edge_index[1]
    dst_pad = jnp.concatenate(
        [dst, jnp.full((_EPAD - E,), N, dtype=jnp.int32)]).reshape(_RPAD, 128)
    ea8 = jnp.concatenate(
        [edge_attr, jnp.ones((E, 1), jnp.float32), jnp.zeros((E, 1), jnp.float32)],
        axis=1)
    ea8_pad = jnp.concatenate([ea8, jnp.zeros((_EPAD - E, 8), jnp.float32)], axis=0)
    zero8 = jnp.zeros((_NP, 8), jnp.float32)
    parts = _easum_sc(dst_pad, ea8_pad, zero8)
    tot = parts[0] + parts[1]
    sums = tot[:N, :6]
    cnt = tot[:N, 6]
    return sums / jnp.clip(cnt, 1.0, None)[:, None]


_ECHUNK = 2000
_EPT = E // _NS
_NCH = _EPT // _ECHUNK


def _make_edge_pass(heads, row, ch, rounds, nn):
    fpt = ch // _NS
    feat = row - 16

    def body(src_hbm, dst_hbm, ae_hbm, xh_hbm, adst_hbm, init_hbm, out_hbm,
             srcb, dstb, aeb, adstb, matchb, gixb, scixb, gbufs, exb, sem, acc):
        c = lax.axis_index("c")
        s = lax.axis_index("s")
        iota = lax.iota(jnp.int32, 16)
        ones = jnp.ones((16,), jnp.int32)
        tailmask = (iota < heads).astype(jnp.float32)
        eh = iota // heads
        hh = iota % heads
        epg = 16 // heads

        def round_body(r, _r):
            k = 2 * r + c
            lo = pl.multiple_of(k * ch, 128)
            pltpu.sync_copy(
                init_hbm.at[pl.ds(pl.multiple_of(lo + s * fpt, 8), fpt)],
                acc.at[pl.ds(pl.multiple_of(s * fpt, 8), fpt)])
            pltpu.sync_copy(adst_hbm.at[pl.ds(pl.multiple_of(lo * heads, 8),
                                              ch * heads)],
                            adstb.at[pl.ds(0, ch * heads)])
            plsc.subcore_barrier()

            def chunk_body(ci, _c):
                base = s * _EPT + ci * _ECHUNK
                pltpu.sync_copy(src_hbm.at[pl.ds(base, _ECHUNK)],
                                srcb.at[pl.ds(0, _ECHUNK)])
                pltpu.sync_copy(dst_hbm.at[pl.ds(base, _ECHUNK)],
                                dstb.at[pl.ds(0, _ECHUNK)])
                pltpu.sync_copy(ae_hbm.at[pl.ds(base * heads, _ECHUNK * heads)],
                                aeb.at[pl.ds(0, _ECHUNK * heads)])
                dstb[pl.ds(_ECHUNK, 16)] = jnp.full((16,), ch, jnp.int32) + lo
                srcb[pl.ds(_ECHUNK, 16)] = jnp.zeros((16,), jnp.int32)
                aeb[pl.ds(_ECHUNK * heads, 16)] = jnp.zeros((16,), jnp.float32)

                def filt(v, cnt):
                    d16 = dstb[pl.ds(v * 16, 16)] - lo
                    m = (d16 >= 0) & (d16 < ch)
                    pos = cnt + plsc.cumsum(ones, mask=m) - 1
                    plsc.store_scatter(matchb, [pos], iota + v * 16, mask=m)
                    return cnt + jnp.sum(m.astype(jnp.int32))

                cnt = lax.fori_loop(0, _ECHUNK // 16, filt, 0)
                kpad = (cnt + 15) & ~15
                plsc.store_scatter(matchb, [cnt + iota],
                                   jnp.full((16,), _ECHUNK, jnp.int32),
                                   mask=iota < (kpad - cnt))

                ngroups = kpad // 16

                def start_g(g, gix, gbf, sm):
                    src16 = plsc.load_gather(
                        srcb, [plsc.load_gather(matchb, [g * 16 + iota])])
                    gix[...] = src16
                    pltpu.async_copy(xh_hbm.at[gix], gbf, sm)

                def do_group(g, gix, gbf, sm):
                    pltpu.make_async_copy(xh_hbm.at[gix], gbf, sm).wait()
                    ids16 = plsc.load_gather(matchb, [g * 16 + iota])
                    dloc16 = plsc.load_gather(dstb, [ids16]) - lo
                    scixb[...] = dloc16
                    for q in range(heads):
                        eq = eh + q * epg
                        idq = plsc.load_gather(matchb, [g * 16 + eq])
                        asrc = plsc.load_gather(gbf, [eq, hh + feat])
                        dq = plsc.load_gather(dstb, [idq]) - lo
                        adst = plsc.load_gather(adstb, [dq * heads + hh])
                        ae = plsc.load_gather(aeb, [idq * heads + hh])
                        a = asrc + adst + ae
                        a = jnp.maximum(a, 0.0) + 0.2 * jnp.minimum(a, 0.0)
                        exb[pl.ds(q * 16, 16)] = jnp.exp(a)
                    for e in range(16):
                        exvec = exb[pl.ds(e * heads, 16)]
                        for j in range(feat // 16):
                            sc = exvec[(j * 16) // 64]
                            gbf[e, pl.ds(j * 16, 16)] = (
                                gbf[e, pl.ds(j * 16, 16)] * sc)
                        gbf[e, pl.ds(feat, 16)] = exvec * tailmask
                    pltpu.sync_copy(gbf, acc.at[scixb], add=True)

                gbufA, gbufB = gbufs.at[0], gbufs.at[1]
                gixA, gixB = gixb.at[0], gixb.at[1]
                semA, semB = sem.at[0], sem.at[1]

                @pl.when(ngroups > 0)
                def _prime():
                    start_g(0, gixA, gbufA, semA)

                def pair(pr, _p):
                    g0 = 2 * pr

                    @pl.when(g0 + 1 < ngroups)
                    def _startB():
                        start_g(g0 + 1, gixB, gbufB, semB)

                    do_group(g0, gixA, gbufA, semA)

                    @pl.when(g0 + 1 < ngroups)
                    def _doB():
                        @pl.when(g0 + 2 < ngroups)
                        def _startA():
                            start_g(g0 + 2, gixA, gbufA, semA)

                        do_group(g0 + 1, gixB, gbufB, semB)

                    return _p

                lax.fori_loop(0, (ngroups + 1) // 2, pair, 0)
                return _c

            lax.fori_loop(0, _NCH, chunk_body, 0)
            plsc.subcore_barrier()
            pltpu.sync_copy(
                acc.at[pl.ds(pl.multiple_of(s * fpt, 8), fpt)],
                out_hbm.at[pl.ds(pl.multiple_of(lo + s * fpt, 8), fpt)])
            plsc.subcore_barrier()
            return _r

        lax.fori_loop(0, rounds, round_body, 0)

    mesh = plsc.VectorSubcoreMesh(core_axis_name="c", subcore_axis_name="s")
    return pl.kernel(
        body,
        out_type=jax.ShapeDtypeStruct((nn, row), jnp.float32),
        mesh=mesh,
        scratch_types=[
            pltpu.VMEM((_ECHUNK + 16,), jnp.int32),
            pltpu.VMEM((_ECHUNK + 16,), jnp.int32),
            pltpu.VMEM((_ECHUNK * heads + 16,), jnp.float32),
            pltpu.VMEM(((ch + 8) * heads,), jnp.float32),
            pltpu.VMEM((_ECHUNK + 16,), jnp.int32),
            pltpu.VMEM((2, 16), jnp.int32),
            pltpu.VMEM((16,), jnp.int32),
            pltpu.VMEM((2, 16, row), jnp.float32),
            pltpu.VMEM((16 * heads + 16,), jnp.float32),
            pltpu.SemaphoreType.DMA((2,)),
            pltpu.VMEM_SHARED((ch + 8, row), jnp.float32),
        ],
        compiler_params=pltpu.CompilerParams(use_tc_tiling_on_sc=False, needs_layout_passes=False),
    )


_CH1, _R1, _NN1 = 5120, 5, 51200
_CH2, _R2, _NN2 = 12544, 2, 50176
_EP1 = _make_edge_pass(H1, H1 * C1 + 16, _CH1, _R1, _NN1)
_EP2 = _make_edge_pass(1, C2 + 16, _CH2, _R2, _NN2)


def _gat_conv_sc(ep, nn, x, src, dst, edge_attr, mean_ea, W, att_src,
                 att_dst, W_e, att_e, bias, heads, out_ch, num_nodes):
    feat = heads * out_ch
    row = feat + 16
    xh = (x @ W)
    xh3 = xh.reshape(num_nodes, heads, out_ch)
    a_src = jnp.sum(xh3 * att_src, axis=-1)
    a_dst = jnp.sum(xh3 * att_dst, axis=-1)
    Ve = (W_e.reshape(-1, heads, out_ch) * att_e).sum(-1)
    ae_flat = (edge_attr @ Ve).reshape(-1)
    a_e_loop = mean_ea @ Ve

    xh_ext = jnp.concatenate(
        [xh, a_src, jnp.zeros((num_nodes, 16 - heads), jnp.float32)], axis=1)
    adst_flat = jnp.concatenate(
        [a_dst.reshape(-1),
         jnp.zeros(((nn - num_nodes) * heads,), jnp.float32)])

    alpha_l = a_src + a_dst + a_e_loop
    alpha_l = jnp.maximum(alpha_l, 0.0) + 0.2 * jnp.minimum(alpha_l, 0.0)
    ex_l = jnp.exp(alpha_l)
    init = jnp.zeros((nn, row), jnp.float32)

    out_ext = ep(src, dst, ae_flat, xh_ext, adst_flat, init)
    num = (out_ext[:num_nodes, :feat].reshape(num_nodes, heads, out_ch)
           + xh3 * ex_l[:, :, None])
    den = out_ext[:num_nodes, feat:feat + heads] + ex_l
    out = num / (den[:, :, None] + 1e-16)
    if heads > 1:
        out = out.reshape(num_nodes, feat)
    else:
        out = out[:, 0]
    return out + bias


def _layer_norm(x, g, b, eps=1e-5):
    mu = x.mean(-1, keepdims=True)
    var = x.var(-1, keepdims=True)
    return (x - mu) / jnp.sqrt(var + eps) * g + b


def kernel(x, edge_index, edge_attr, W1, as1, ad1, We1, ae1, b1, g1, be1,
           W2, as2, ad2, We2, ae2, b2, g2, be2):
    src, dst = edge_index[0], edge_index[1]
    mean_ea = _self_loop_mean(edge_index, edge_attr)
    h = _gat_conv_sc(_EP1, _NN1, x, src, dst, edge_attr, mean_ea, W1, as1,
                     ad1, We1, ae1, b1, H1, C1, N)
    h = _layer_norm(h, g1, be1)
    h = jax.nn.relu(h)
    h = _gat_conv_sc(_EP2, _NN2, h, src, dst, edge_attr, mean_ea, W2, as2,
                     ad2, We2, ae2, b2, 1, C2, N)
    h = _layer_norm(h, g2, be2)
    return h


# speedup vs baseline: 28.0423x; 1.0563x over previous
import functools

import jax
import jax.numpy as jnp
from jax import lax
from jax.experimental import pallas as pl
from jax.experimental.pallas import tpu as pltpu
from jax.experimental.pallas import tpu_sc as plsc

N = 50000
E = 800000
H1 = 4
C1 = 64
C2 = 64

_NC = 2
_NS = 16
_NW = _NC * _NS


_ROWS = (E + 127) // 128
_RPW = 200
_RPAD = _RPW * _NW
_EPAD = _RPAD * 128
_NP = N + 48
_CHUNK = 8
_NCHUNK = _RPW // _CHUNK


def _easum_body(dst_hbm, ea8_hbm, zero_hbm, out_hbm, dstbuf, eabuf, acc):
    c = lax.axis_index("c")
    s = lax.axis_index("s")
    w = c * _NS + s
    zrows = _NP // _NS
    pltpu.sync_copy(zero_hbm.at[pl.ds(s * zrows, zrows)],
                    acc.at[pl.ds(s * zrows, zrows)])
    plsc.subcore_barrier()

    def chunk_body(i, _):
        row0 = w * _RPW + i * _CHUNK
        pltpu.sync_copy(dst_hbm.at[pl.ds(row0, _CHUNK)], dstbuf)
        pltpu.sync_copy(ea8_hbm.at[pl.ds(row0 * 128, _CHUNK * 128)], eabuf)
        for j in range(_CHUNK):
            pltpu.sync_copy(eabuf.at[pl.ds(j * 128, 128)],
                            acc.at[dstbuf.at[j]], add=True)
        return 0

    lax.fori_loop(0, _NCHUNK, chunk_body, 0)
    plsc.subcore_barrier()
    frows = _NP // _NS
    pltpu.sync_copy(acc.at[pl.ds(s * frows, frows)],
                    out_hbm.at[c, pl.ds(s * frows, frows)])


@jax.jit
def _easum_sc(dst_pad, ea8_pad, zero8):
    mesh = plsc.VectorSubcoreMesh(core_axis_name="c", subcore_axis_name="s")
    f = pl.kernel(
        _easum_body,
        out_type=jax.ShapeDtypeStruct((_NC, _NP, 8), jnp.float32),
        mesh=mesh,
        scratch_types=[
            pltpu.VMEM((_CHUNK, 128), jnp.int32),
            pltpu.VMEM((_CHUNK * 128, 8), jnp.float32),
            pltpu.VMEM_SHARED((_NP, 8), jnp.float32),
        ],
        compiler_params=pltpu.CompilerParams(use_tc_tiling_on_sc=False, needs_layout_passes=False),
    )
    return f(dst_pad, ea8_pad, zero8)


def _easum_parts(edge_index, edge_attr):
    dst = edge_index[1]
    dst_pad = jnp.concatenate(
        [dst, jnp.full((_EPAD - E,), N, dtype=jnp.int32)]).reshape(_RPAD, 128)
    ea8 = jnp.concatenate(
        [edge_attr, jnp.ones((E, 1), jnp.float32), jnp.zeros((E, 1), jnp.float32)],
        axis=1)
    ea8_pad = jnp.concatenate([ea8, jnp.zeros((_EPAD - E, 8), jnp.float32)], axis=0)
    zero8 = jnp.zeros((_NP, 8), jnp.float32)
    return _easum_sc(dst_pad, ea8_pad, zero8)


_ECHUNK = 2000
_EPT = E // _NS
_NCH = _EPT // _ECHUNK


def _make_edge_pass(heads, row, ch, rounds, nn):
    fpt = ch // _NS
    feat = row - 16

    def body(src_hbm, dst_hbm, ae_hbm, xh_hbm, adst_hbm, init_hbm, out_hbm,
             srcb, dstb, aeb, adstb, matchb, gixb, scixb, gbufs, exb, sem, acc):
        c = lax.axis_index("c")
        s = lax.axis_index("s")
        iota = lax.iota(jnp.int32, 16)
        ones = jnp.ones((16,), jnp.int32)
        tailmask = (iota < heads).astype(jnp.float32)
        eh = iota // heads
        hh = iota % heads
        epg = 16 // heads

        def round_body(r, _r):
            k = 2 * r + c
            lo = pl.multiple_of(k * ch, 128)
            pltpu.sync_copy(
                init_hbm.at[pl.ds(pl.multiple_of(lo + s * fpt, 8), fpt)],
                acc.at[pl.ds(pl.multiple_of(s * fpt, 8), fpt)])
            pltpu.sync_copy(adst_hbm.at[pl.ds(pl.multiple_of(lo * heads, 8),
                                              ch * heads)],
                            adstb.at[pl.ds(0, ch * heads)])
            plsc.subcore_barrier()

            def chunk_body(ci, _c):
                base = s * _EPT + ci * _ECHUNK
                pltpu.sync_copy(src_hbm.at[pl.ds(base, _ECHUNK)],
                                srcb.at[pl.ds(0, _ECHUNK)])
                pltpu.sync_copy(dst_hbm.at[pl.ds(base, _ECHUNK)],
                                dstb.at[pl.ds(0, _ECHUNK)])
                pltpu.sync_copy(ae_hbm.at[pl.ds(base * heads, _ECHUNK * heads)],
                                aeb.at[pl.ds(0, _ECHUNK * heads)])
                dstb[pl.ds(_ECHUNK, 16)] = jnp.full((16,), ch, jnp.int32) + lo
                srcb[pl.ds(_ECHUNK, 16)] = jnp.zeros((16,), jnp.int32)
                aeb[pl.ds(_ECHUNK * heads, 16)] = jnp.zeros((16,), jnp.float32)

                def filt(v, cnt):
                    d16 = dstb[pl.ds(v * 16, 16)] - lo
                    m = (d16 >= 0) & (d16 < ch)
                    pos = cnt + plsc.cumsum(ones, mask=m) - 1
                    plsc.store_scatter(matchb, [pos], iota + v * 16, mask=m)
                    return cnt + jnp.sum(m.astype(jnp.int32))

                cnt = lax.fori_loop(0, _ECHUNK // 16, filt, 0)
                kpad = (cnt + 15) & ~15
                plsc.store_scatter(matchb, [cnt + iota],
                                   jnp.full((16,), _ECHUNK, jnp.int32),
                                   mask=iota < (kpad - cnt))

                ngroups = kpad // 16

                def start_g(g, gix, gbf, sm):
                    src16 = plsc.load_gather(
                        srcb, [plsc.load_gather(matchb, [g * 16 + iota])])
                    gix[...] = src16
                    pltpu.async_copy(xh_hbm.at[gix], gbf, sm)

                def do_group(g, gix, gbf, sm):
                    pltpu.make_async_copy(xh_hbm.at[gix], gbf, sm).wait()
                    ids16 = plsc.load_gather(matchb, [g * 16 + iota])
                    dloc16 = plsc.load_gather(dstb, [ids16]) - lo
                    scixb[...] = dloc16
                    for q in range(heads):
                        eq = eh + q * epg
                        idq = plsc.load_gather(matchb, [g * 16 + eq])
                        asrc = plsc.load_gather(gbf, [eq, hh + feat])
                        dq = plsc.load_gather(dstb, [idq]) - lo
                        adst = plsc.load_gather(adstb, [dq * heads + hh])
                        ae = plsc.load_gather(aeb, [idq * heads + hh])
                        a = asrc + adst + ae
                        a = jnp.maximum(a, 0.0) + 0.2 * jnp.minimum(a, 0.0)
                        exb[pl.ds(q * 16, 16)] = jnp.exp(a)
                    for e in range(16):
                        exvec = exb[pl.ds(e * heads, 16)]
                        for j in range(feat // 16):
                            sc = exvec[(j * 16) // 64]
                            gbf[e, pl.ds(j * 16, 16)] = (
                                gbf[e, pl.ds(j * 16, 16)] * sc)
                        gbf[e, pl.ds(feat, 16)] = exvec * tailmask
                    pltpu.sync_copy(gbf, acc.at[scixb], add=True)

                gbufA, gbufB = gbufs.at[0], gbufs.at[1]
                gixA, gixB = gixb.at[0], gixb.at[1]
                semA, semB = sem.at[0], sem.at[1]

                @pl.when(ngroups > 0)
                def _prime():
                    start_g(0, gixA, gbufA, semA)

                def pair(pr, _p):
                    g0 = 2 * pr

                    @pl.when(g0 + 1 < ngroups)
                    def _startB():
                        start_g(g0 + 1, gixB, gbufB, semB)

                    do_group(g0, gixA, gbufA, semA)

                    @pl.when(g0 + 1 < ngroups)
                    def _doB():
                        @pl.when(g0 + 2 < ngroups)
                        def _startA():
                            start_g(g0 + 2, gixA, gbufA, semA)

                        do_group(g0 + 1, gixB, gbufB, semB)

                    return _p

                lax.fori_loop(0, (ngroups + 1) // 2, pair, 0)
                return _c

            lax.fori_loop(0, _NCH, chunk_body, 0)
            plsc.subcore_barrier()
            pltpu.sync_copy(
                acc.at[pl.ds(pl.multiple_of(s * fpt, 8), fpt)],
                out_hbm.at[pl.ds(pl.multiple_of(lo + s * fpt, 8), fpt)])
            plsc.subcore_barrier()
            return _r

        lax.fori_loop(0, rounds, round_body, 0)

    mesh = plsc.VectorSubcoreMesh(core_axis_name="c", subcore_axis_name="s")
    return pl.kernel(
        body,
        out_type=jax.ShapeDtypeStruct((nn, row), jnp.float32),
        mesh=mesh,
        scratch_types=[
            pltpu.VMEM((_ECHUNK + 16,), jnp.int32),
            pltpu.VMEM((_ECHUNK + 16,), jnp.int32),
            pltpu.VMEM((_ECHUNK * heads + 16,), jnp.float32),
            pltpu.VMEM(((ch + 8) * heads,), jnp.float32),
            pltpu.VMEM((_ECHUNK + 16,), jnp.int32),
            pltpu.VMEM((2, 16), jnp.int32),
            pltpu.VMEM((16,), jnp.int32),
            pltpu.VMEM((2, 16, row), jnp.float32),
            pltpu.VMEM((16 * heads + 16,), jnp.float32),
            pltpu.SemaphoreType.DMA((2,)),
            pltpu.VMEM_SHARED((ch + 8, row), jnp.float32),
        ],
        compiler_params=pltpu.CompilerParams(use_tc_tiling_on_sc=False, needs_layout_passes=False),
    )


_CH1, _R1, _NN1 = 5120, 5, 51200
_CH2, _R2, _NN2 = 12544, 2, 50176
_EP1 = _make_edge_pass(H1, H1 * C1 + 16, _CH1, _R1, _NN1)
_EP2 = _make_edge_pass(1, C2 + 16, _CH2, _R2, _NN2)


_BLK = 1000
_EBLK = 10000


def _prol_body(x_ref, w_ref, as_ref, ad_ref, xe_ref, adst_ref):
    xh = jnp.dot(x_ref[...], w_ref[...], preferred_element_type=jnp.float32)
    asrc = jnp.dot(xh, as_ref[...], preferred_element_type=jnp.float32)
    h = as_ref.shape[1]
    xe_ref[...] = jnp.concatenate(
        [xh, asrc, jnp.zeros((xh.shape[0], 16 - h), jnp.float32)], axis=1)
    adst_ref[...] = jnp.dot(xh, ad_ref[...], preferred_element_type=jnp.float32)


def _prologue_tc(x, W, AS, AD):
    din, f = W.shape
    h = AS.shape[1]
    row = f + 16
    return pl.pallas_call(
        _prol_body,
        grid=(N // _BLK,),
        in_specs=[pl.BlockSpec((_BLK, din), lambda i: (i, 0)),
                  pl.BlockSpec((din, f), lambda i: (0, 0)),
                  pl.BlockSpec((f, h), lambda i: (0, 0)),
                  pl.BlockSpec((f, h), lambda i: (0, 0))],
        out_specs=[pl.BlockSpec((_BLK, row), lambda i: (i, 0)),
                   pl.BlockSpec((_BLK, h), lambda i: (i, 0))],
        out_shape=[jax.ShapeDtypeStruct((N, row), jnp.float32),
                   jax.ShapeDtypeStruct((N, h), jnp.float32)],
    )(x, W, AS, AD)


def _ae_body(ea_ref, v1_ref, v2_ref, o1_ref, o2_ref):
    ea = ea_ref[...]
    o1_ref[...] = jnp.dot(ea, v1_ref[...], preferred_element_type=jnp.float32)
    o2_ref[...] = jnp.dot(ea, v2_ref[...], preferred_element_type=jnp.float32)


def _ae_edge_tc(ea, Ve1, Ve2):
    return pl.pallas_call(
        _ae_body,
        grid=(E // _EBLK,),
        in_specs=[pl.BlockSpec((_EBLK, 6), lambda i: (i, 0)),
                  pl.BlockSpec((6, H1), lambda i: (0, 0)),
                  pl.BlockSpec((6, 1), lambda i: (0, 0))],
        out_specs=[pl.BlockSpec((_EBLK, H1), lambda i: (i, 0)),
                   pl.BlockSpec((_EBLK, 1), lambda i: (i, 0))],
        out_shape=[jax.ShapeDtypeStruct((E, H1), jnp.float32),
                   jax.ShapeDtypeStruct((E, 1), jnp.float32)],
    )(ea, Ve1, Ve2)


def _aeloop_body(p_ref, v1_ref, v2_ref, o1_ref, o2_ref):
    p = p_ref[0] + p_ref[1]
    mean6 = p[:, :6] / jnp.clip(p[:, 6:7], 1.0, None)
    o1_ref[...] = jnp.dot(mean6, v1_ref[...], preferred_element_type=jnp.float32)
    o2_ref[...] = jnp.dot(mean6, v2_ref[...], preferred_element_type=jnp.float32)


def _aeloop_tc(parts, Ve1, Ve2):
    return pl.pallas_call(
        _aeloop_body,
        grid=(N // _BLK,),
        in_specs=[pl.BlockSpec((2, _BLK, 8), lambda i: (0, i, 0)),
                  pl.BlockSpec((6, H1), lambda i: (0, 0)),
                  pl.BlockSpec((6, 1), lambda i: (0, 0))],
        out_specs=[pl.BlockSpec((_BLK, H1), lambda i: (i, 0)),
                   pl.BlockSpec((_BLK, 1), lambda i: (i, 0))],
        out_shape=[jax.ShapeDtypeStruct((N, H1), jnp.float32),
                   jax.ShapeDtypeStruct((N, 1), jnp.float32)],
    )(parts, Ve1, Ve2)


def _make_epi_body(heads, out_ch, relu):
    feat = heads * out_ch

    def body(oe_ref, xe_ref, adst_ref, ael_ref, b_ref, g_ref, be_ref, o_ref):
        xe = xe_ref[...]
        oe = oe_ref[...]
        asrc = xe[:, feat:feat + heads]
        alpha = asrc + adst_ref[...] + ael_ref[...]
        alpha = jnp.maximum(alpha, 0.0) + 0.2 * jnp.minimum(alpha, 0.0)
        exl = jnp.exp(alpha)
        parts = []
        for hh in range(heads):
            xhh = xe[:, hh * out_ch:(hh + 1) * out_ch]
            numh = oe[:, hh * out_ch:(hh + 1) * out_ch] + xhh * exl[:, hh:hh + 1]
            denh = oe[:, feat + hh:feat + hh + 1] + exl[:, hh:hh + 1]
            parts.append(numh / (denh + 1e-16))
        o = jnp.concatenate(parts, axis=1) if heads > 1 else parts[0]
        o = o + b_ref[...]
        mu = jnp.mean(o, axis=-1, keepdims=True)
        var = jnp.mean((o - mu) ** 2, axis=-1, keepdims=True)
        o = (o - mu) / jnp.sqrt(var + 1e-5) * g_ref[...] + be_ref[...]
        if relu:
            o = jnp.maximum(o, 0.0)
        o_ref[...] = o

    return body


def _epilogue_tc(oe, xe, adst, ael, b, g, be, heads, out_ch, relu):
    feat = heads * out_ch
    row = feat + 16
    nn = oe.shape[0]
    return pl.pallas_call(
        _make_epi_body(heads, out_ch, relu),
        grid=(N // _BLK,),
        in_specs=[pl.BlockSpec((_BLK, row), lambda i: (i, 0)),
                  pl.BlockSpec((_BLK, row), lambda i: (i, 0)),
                  pl.BlockSpec((_BLK, heads), lambda i: (i, 0)),
                  pl.BlockSpec((_BLK, heads), lambda i: (i, 0)),
                  pl.BlockSpec((feat,), lambda i: (0,)),
                  pl.BlockSpec((feat,), lambda i: (0,)),
                  pl.BlockSpec((feat,), lambda i: (0,))],
        out_specs=pl.BlockSpec((_BLK, feat), lambda i: (i, 0)),
        out_shape=jax.ShapeDtypeStruct((N, feat), jnp.float32),
    )(oe, xe, adst, ael, b, g, be)


def _fold(att):
    h = att.shape[0]
    return (jnp.eye(h, dtype=jnp.float32)[:, None, :]
            * att[:, :, None]).reshape(-1, h)


def kernel(x, edge_index, edge_attr, W1, as1, ad1, We1, ae1, b1, g1, be1,
           W2, as2, ad2, We2, ae2, b2, g2, be2):
    src, dst = edge_index[0], edge_index[1]
    Ve1 = (We1.reshape(6, H1, C1) * ae1).sum(-1)
    Ve2 = (We2.reshape(6, 1, C2) * ae2).sum(-1)

    parts = _easum_parts(edge_index, edge_attr)
    ael1, ael2 = _aeloop_tc(parts, Ve1, Ve2)
    aee1, aee2 = _ae_edge_tc(edge_attr, Ve1, Ve2)

    xe1, adst1 = _prologue_tc(x, W1, _fold(as1), _fold(ad1))
    adst1_flat = jnp.concatenate(
        [adst1.reshape(-1), jnp.zeros(((_NN1 - N) * H1,), jnp.float32)])
    oe1 = _EP1(src, dst, aee1.reshape(-1), xe1, adst1_flat,
               jnp.zeros((_NN1, H1 * C1 + 16), jnp.float32))
    h1 = _epilogue_tc(oe1, xe1, adst1, ael1, b1, g1, be1, H1, C1, True)

    xe2, adst2 = _prologue_tc(h1, W2, _fold(as2), _fold(ad2))
    adst2_flat = jnp.concatenate(
        [adst2.reshape(-1), jnp.zeros((_NN2 - N,), jnp.float32)])
    oe2 = _EP2(src, dst, aee2.reshape(-1), xe2, adst2_flat,
               jnp.zeros((_NN2, C2 + 16), jnp.float32))
    return _epilogue_tc(oe2, xe2, adst2, ael2, b2, g2, be2, 1, C2, False)


# speedup vs baseline: 28.0931x; 1.0018x over previous
import functools

import jax
import jax.numpy as jnp
from jax import lax
from jax.experimental import pallas as pl
from jax.experimental.pallas import tpu as pltpu
from jax.experimental.pallas import tpu_sc as plsc

N = 50000
E = 800000
H1 = 4
C1 = 64
C2 = 64

_NC = 2
_NS = 16
_NW = _NC * _NS


_ROWS = (E + 127) // 128
_RPW = 200
_RPAD = _RPW * _NW
_EPAD = _RPAD * 128
_NP = N + 48
_CHUNK = 8
_NCHUNK = _RPW // _CHUNK


def _easum_body(dst_hbm, ea8_hbm, zero_hbm, out_hbm, dstbuf, eabuf, acc):
    c = lax.axis_index("c")
    s = lax.axis_index("s")
    w = c * _NS + s
    zrows = _NP // _NS
    pltpu.sync_copy(zero_hbm.at[pl.ds(s * zrows, zrows)],
                    acc.at[pl.ds(s * zrows, zrows)])
    plsc.subcore_barrier()

    def chunk_body(i, _):
        row0 = w * _RPW + i * _CHUNK
        pltpu.sync_copy(dst_hbm.at[pl.ds(row0, _CHUNK)], dstbuf)
        pltpu.sync_copy(ea8_hbm.at[pl.ds(row0 * 128, _CHUNK * 128)], eabuf)
        for j in range(_CHUNK):
            pltpu.sync_copy(eabuf.at[pl.ds(j * 128, 128)],
                            acc.at[dstbuf.at[j]], add=True)
        return 0

    lax.fori_loop(0, _NCHUNK, chunk_body, 0)
    plsc.subcore_barrier()
    frows = _NP // _NS
    pltpu.sync_copy(acc.at[pl.ds(s * frows, frows)],
                    out_hbm.at[c, pl.ds(s * frows, frows)])


@jax.jit
def _easum_sc(dst_pad, ea8_pad, zero8):
    mesh = plsc.VectorSubcoreMesh(core_axis_name="c", subcore_axis_name="s")
    f = pl.kernel(
        _easum_body,
        out_type=jax.ShapeDtypeStruct((_NC, _NP, 8), jnp.float32),
        mesh=mesh,
        scratch_types=[
            pltpu.VMEM((_CHUNK, 128), jnp.int32),
            pltpu.VMEM((_CHUNK * 128, 8), jnp.float32),
            pltpu.VMEM_SHARED((_NP, 8), jnp.float32),
        ],
        compiler_params=pltpu.CompilerParams(use_tc_tiling_on_sc=False, needs_layout_passes=False),
    )
    return f(dst_pad, ea8_pad, zero8)


def _easum_parts(edge_index, edge_attr):
    dst = edge_index[1]
    dst_pad = jnp.concatenate(
        [dst, jnp.full((_EPAD - E,), N, dtype=jnp.int32)]).reshape(_RPAD, 128)
    ea8 = jnp.concatenate(
        [edge_attr, jnp.ones((E, 1), jnp.float32), jnp.zeros((E, 1), jnp.float32)],
        axis=1)
    ea8_pad = jnp.concatenate([ea8, jnp.zeros((_EPAD - E, 8), jnp.float32)], axis=0)
    zero8 = jnp.zeros((_NP, 8), jnp.float32)
    return _easum_sc(dst_pad, ea8_pad, zero8)


_ECHUNK = 2000
_EPT = E // _NS
_NCH = _EPT // _ECHUNK


def _make_edge_pass(heads, row, ch, rounds, nn):
    fpt = ch // _NS
    feat = row - 16

    def body(src_hbm, dst_hbm, ae_hbm, xh_hbm, adst_hbm, init_hbm, out_hbm,
             srcb, dstb, aeb, adstb, matchb, gixb, scixb, gbufs, exb, sem, ssem, acc):
        c = lax.axis_index("c")
        s = lax.axis_index("s")
        iota = lax.iota(jnp.int32, 16)
        ones = jnp.ones((16,), jnp.int32)
        tailmask = (iota < heads).astype(jnp.float32)
        eh = iota // heads
        hh = iota % heads
        epg = 16 // heads

        def round_body(r, _r):
            k = 2 * r + c
            lo = pl.multiple_of(k * ch, 128)
            pltpu.sync_copy(
                init_hbm.at[pl.ds(pl.multiple_of(lo + s * fpt, 8), fpt)],
                acc.at[pl.ds(pl.multiple_of(s * fpt, 8), fpt)])
            pltpu.sync_copy(adst_hbm.at[pl.ds(pl.multiple_of(lo * heads, 8),
                                              ch * heads)],
                            adstb.at[pl.ds(0, ch * heads)])
            plsc.subcore_barrier()

            def chunk_body(ci, _c):
                base = s * _EPT + ci * _ECHUNK
                pltpu.sync_copy(src_hbm.at[pl.ds(base, _ECHUNK)],
                                srcb.at[pl.ds(0, _ECHUNK)])
                pltpu.sync_copy(dst_hbm.at[pl.ds(base, _ECHUNK)],
                                dstb.at[pl.ds(0, _ECHUNK)])
                pltpu.sync_copy(ae_hbm.at[pl.ds(base * heads, _ECHUNK * heads)],
                                aeb.at[pl.ds(0, _ECHUNK * heads)])
                dstb[pl.ds(_ECHUNK, 16)] = jnp.full((16,), ch, jnp.int32) + lo
                srcb[pl.ds(_ECHUNK, 16)] = jnp.zeros((16,), jnp.int32)
                aeb[pl.ds(_ECHUNK * heads, 16)] = jnp.zeros((16,), jnp.float32)

                def filt(v, cnt):
                    d16 = dstb[pl.ds(v * 16, 16)] - lo
                    m = (d16 >= 0) & (d16 < ch)
                    pos = cnt + plsc.cumsum(ones, mask=m) - 1
                    plsc.store_scatter(matchb, [pos], iota + v * 16, mask=m)
                    return cnt + jnp.sum(m.astype(jnp.int32))

                cnt = lax.fori_loop(0, _ECHUNK // 16, filt, 0)
                kpad = (cnt + 15) & ~15
                plsc.store_scatter(matchb, [cnt + iota],
                                   jnp.full((16,), _ECHUNK, jnp.int32),
                                   mask=iota < (kpad - cnt))

                ngroups = kpad // 16

                def start_g(g, gix, gbf, sm, scix, ssm):
                    @pl.when(g >= 2)
                    def _drain():
                        pltpu.make_async_copy(gbf, acc.at[scix], ssm).wait()
                    src16 = plsc.load_gather(
                        srcb, [plsc.load_gather(matchb, [g * 16 + iota])])
                    gix[...] = src16
                    pltpu.async_copy(xh_hbm.at[gix], gbf, sm)

                def do_group(g, gix, gbf, sm, scix, ssm):
                    pltpu.make_async_copy(xh_hbm.at[gix], gbf, sm).wait()
                    ids16 = plsc.load_gather(matchb, [g * 16 + iota])
                    dloc16 = plsc.load_gather(dstb, [ids16]) - lo
                    scix[...] = dloc16
                    for q in range(heads):
                        eq = eh + q * epg
                        idq = plsc.load_gather(matchb, [g * 16 + eq])
                        asrc = plsc.load_gather(gbf, [eq, hh + feat])
                        dq = plsc.load_gather(dstb, [idq]) - lo
                        adst = plsc.load_gather(adstb, [dq * heads + hh])
                        ae = plsc.load_gather(aeb, [idq * heads + hh])
                        a = asrc + adst + ae
                        a = jnp.maximum(a, 0.0) + 0.2 * jnp.minimum(a, 0.0)
                        exb[pl.ds(q * 16, 16)] = jnp.exp(a)
                    for e in range(16):
                        exvec = exb[pl.ds(e * heads, 16)]
                        for j in range(feat // 16):
                            sc = exvec[(j * 16) // 64]
                            gbf[e, pl.ds(j * 16, 16)] = (
                                gbf[e, pl.ds(j * 16, 16)] * sc)
                        gbf[e, pl.ds(feat, 16)] = exvec * tailmask
                    pltpu.async_copy(gbf, acc.at[scix], ssm, add=True)

                gbufA, gbufB = gbufs.at[0], gbufs.at[1]
                gixA, gixB = gixb.at[0], gixb.at[1]
                scixA, scixB = scixb.at[0], scixb.at[1]
                semA, semB = sem.at[0], sem.at[1]
                ssemA, ssemB = ssem.at[0], ssem.at[1]

                @pl.when(ngroups > 0)
                def _prime():
                    start_g(0, gixA, gbufA, semA, scixA, ssemA)

                def pair(pr, _p):
                    g0 = 2 * pr

                    @pl.when(g0 + 1 < ngroups)
                    def _startB():
                        start_g(g0 + 1, gixB, gbufB, semB, scixB, ssemB)

                    do_group(g0, gixA, gbufA, semA, scixA, ssemA)

                    @pl.when(g0 + 1 < ngroups)
                    def _doB():
                        @pl.when(g0 + 2 < ngroups)
                        def _startA():
                            start_g(g0 + 2, gixA, gbufA, semA, scixA, ssemA)

                        do_group(g0 + 1, gixB, gbufB, semB, scixB, ssemB)

                    return _p

                lax.fori_loop(0, (ngroups + 1) // 2, pair, 0)
                @pl.when(ngroups > 0)
                def _drain_last():
                    pl2 = (ngroups - 1) & 1

                    @pl.when(pl2 == 0)
                    def _dA():
                        pltpu.make_async_copy(gbufA, acc.at[scixA], ssemA).wait()

                    @pl.when(pl2 == 1)
                    def _dB():
                        pltpu.make_async_copy(gbufB, acc.at[scixB], ssemB).wait()

                @pl.when(ngroups > 1)
                def _drain_prev():
                    pl3 = (ngroups - 2) & 1

                    @pl.when(pl3 == 0)
                    def _dA2():
                        pltpu.make_async_copy(gbufA, acc.at[scixA], ssemA).wait()

                    @pl.when(pl3 == 1)
                    def _dB2():
                        pltpu.make_async_copy(gbufB, acc.at[scixB], ssemB).wait()

                return _c

            lax.fori_loop(0, _NCH, chunk_body, 0)
            plsc.subcore_barrier()
            pltpu.sync_copy(
                acc.at[pl.ds(pl.multiple_of(s * fpt, 8), fpt)],
                out_hbm.at[pl.ds(pl.multiple_of(lo + s * fpt, 8), fpt)])
            plsc.subcore_barrier()
            return _r

        lax.fori_loop(0, rounds, round_body, 0)

    mesh = plsc.VectorSubcoreMesh(core_axis_name="c", subcore_axis_name="s")
    return pl.kernel(
        body,
        out_type=jax.ShapeDtypeStruct((nn, row), jnp.float32),
        mesh=mesh,
        scratch_types=[
            pltpu.VMEM((_ECHUNK + 16,), jnp.int32),
            pltpu.VMEM((_ECHUNK + 16,), jnp.int32),
            pltpu.VMEM((_ECHUNK * heads + 16,), jnp.float32),
            pltpu.VMEM(((ch + 8) * heads,), jnp.float32),
            pltpu.VMEM((_ECHUNK + 16,), jnp.int32),
            pltpu.VMEM((2, 16), jnp.int32),
            pltpu.VMEM((2, 16), jnp.int32),
            pltpu.VMEM((2, 16, row), jnp.float32),
            pltpu.VMEM((16 * heads + 16,), jnp.float32),
            pltpu.SemaphoreType.DMA((2,)),
            pltpu.SemaphoreType.DMA((2,)),
            pltpu.VMEM_SHARED((ch + 8, row), jnp.float32),
        ],
        compiler_params=pltpu.CompilerParams(use_tc_tiling_on_sc=False, needs_layout_passes=False),
    )


_CH1, _R1, _NN1 = 5120, 5, 51200
_CH2, _R2, _NN2 = 12544, 2, 50176
_EP1 = _make_edge_pass(H1, H1 * C1 + 16, _CH1, _R1, _NN1)
_EP2 = _make_edge_pass(1, C2 + 16, _CH2, _R2, _NN2)


_BLK = 1000
_EBLK = 10000


def _prol_body(x_ref, w_ref, as_ref, ad_ref, xe_ref, adst_ref):
    xh = jnp.dot(x_ref[...], w_ref[...], preferred_element_type=jnp.float32)
    asrc = jnp.dot(xh, as_ref[...], preferred_element_type=jnp.float32)
    h = as_ref.shape[1]
    xe_ref[...] = jnp.concatenate(
        [xh, asrc, jnp.zeros((xh.shape[0], 16 - h), jnp.float32)], axis=1)
    adst_ref[...] = jnp.dot(xh, ad_ref[...], preferred_element_type=jnp.float32)


def _prologue_tc(x, W, AS, AD):
    din, f = W.shape
    h = AS.shape[1]
    row = f + 16
    return pl.pallas_call(
        _prol_body,
        grid=(N // _BLK,),
        in_specs=[pl.BlockSpec((_BLK, din), lambda i: (i, 0)),
                  pl.BlockSpec((din, f), lambda i: (0, 0)),
                  pl.BlockSpec((f, h), lambda i: (0, 0)),
                  pl.BlockSpec((f, h), lambda i: (0, 0))],
        out_specs=[pl.BlockSpec((_BLK, row), lambda i: (i, 0)),
                   pl.BlockSpec((_BLK, h), lambda i: (i, 0))],
        out_shape=[jax.ShapeDtypeStruct((N, row), jnp.float32),
                   jax.ShapeDtypeStruct((N, h), jnp.float32)],
    )(x, W, AS, AD)


def _ae_body(ea_ref, v1_ref, v2_ref, o1_ref, o2_ref):
    ea = ea_ref[...]
    o1_ref[...] = jnp.dot(ea, v1_ref[...], preferred_element_type=jnp.float32)
    o2_ref[...] = jnp.dot(ea, v2_ref[...], preferred_element_type=jnp.float32)


def _ae_edge_tc(ea, Ve1, Ve2):
    return pl.pallas_call(
        _ae_body,
        grid=(E // _EBLK,),
        in_specs=[pl.BlockSpec((_EBLK, 6), lambda i: (i, 0)),
                  pl.BlockSpec((6, H1), lambda i: (0, 0)),
                  pl.BlockSpec((6, 1), lambda i: (0, 0))],
        out_specs=[pl.BlockSpec((_EBLK, H1), lambda i: (i, 0)),
                   pl.BlockSpec((_EBLK, 1), lambda i: (i, 0))],
        out_shape=[jax.ShapeDtypeStruct((E, H1), jnp.float32),
                   jax.ShapeDtypeStruct((E, 1), jnp.float32)],
    )(ea, Ve1, Ve2)


def _aeloop_body(p_ref, v1_ref, v2_ref, o1_ref, o2_ref):
    p = p_ref[0] + p_ref[1]
    mean6 = p[:, :6] / jnp.clip(p[:, 6:7], 1.0, None)
    o1_ref[...] = jnp.dot(mean6, v1_ref[...], preferred_element_type=jnp.float32)
    o2_ref[...] = jnp.dot(mean6, v2_ref[...], preferred_element_type=jnp.float32)


def _aeloop_tc(parts, Ve1, Ve2):
    return pl.pallas_call(
        _aeloop_body,
        grid=(N // _BLK,),
        in_specs=[pl.BlockSpec((2, _BLK, 8), lambda i: (0, i, 0)),
                  pl.BlockSpec((6, H1), lambda i: (0, 0)),
                  pl.BlockSpec((6, 1), lambda i: (0, 0))],
        out_specs=[pl.BlockSpec((_BLK, H1), lambda i: (i, 0)),
                   pl.BlockSpec((_BLK, 1), lambda i: (i, 0))],
        out_shape=[jax.ShapeDtypeStruct((N, H1), jnp.float32),
                   jax.ShapeDtypeStruct((N, 1), jnp.float32)],
    )(parts, Ve1, Ve2)


def _make_epi_body(heads, out_ch, relu):
    feat = heads * out_ch

    def body(oe_ref, xe_ref, adst_ref, ael_ref, b_ref, g_ref, be_ref, o_ref):
        xe = xe_ref[...]
        oe = oe_ref[...]
        asrc = xe[:, feat:feat + heads]
        alpha = asrc + adst_ref[...] + ael_ref[...]
        alpha = jnp.maximum(alpha, 0.0) + 0.2 * jnp.minimum(alpha, 0.0)
        exl = jnp.exp(alpha)
        parts = []
        for hh in range(heads):
            xhh = xe[:, hh * out_ch:(hh + 1) * out_ch]
            numh = oe[:, hh * out_ch:(hh + 1) * out_ch] + xhh * exl[:, hh:hh + 1]
            denh = oe[:, feat + hh:feat + hh + 1] + exl[:, hh:hh + 1]
            parts.append(numh / (denh + 1e-16))
        o = jnp.concatenate(parts, axis=1) if heads > 1 else parts[0]
        o = o + b_ref[...]
        mu = jnp.mean(o, axis=-1, keepdims=True)
        var = jnp.mean((o - mu) ** 2, axis=-1, keepdims=True)
        o = (o - mu) / jnp.sqrt(var + 1e-5) * g_ref[...] + be_ref[...]
        if relu:
            o = jnp.maximum(o, 0.0)
        o_ref[...] = o

    return body


def _epilogue_tc(oe, xe, adst, ael, b, g, be, heads, out_ch, relu):
    feat = heads * out_ch
    row = feat + 16
    nn = oe.shape[0]
    return pl.pallas_call(
        _make_epi_body(heads, out_ch, relu),
        grid=(N // _BLK,),
        in_specs=[pl.BlockSpec((_BLK, row), lambda i: (i, 0)),
                  pl.BlockSpec((_BLK, row), lambda i: (i, 0)),
                  pl.BlockSpec((_BLK, heads), lambda i: (i, 0)),
                  pl.BlockSpec((_BLK, heads), lambda i: (i, 0)),
                  pl.BlockSpec((feat,), lambda i: (0,)),
                  pl.BlockSpec((feat,), lambda i: (0,)),
                  pl.BlockSpec((feat,), lambda i: (0,))],
        out_specs=pl.BlockSpec((_BLK, feat), lambda i: (i, 0)),
        out_shape=jax.ShapeDtypeStruct((N, feat), jnp.float32),
    )(oe, xe, adst, ael, b, g, be)


def _fold(att):
    h = att.shape[0]
    return (jnp.eye(h, dtype=jnp.float32)[:, None, :]
            * att[:, :, None]).reshape(-1, h)


def kernel(x, edge_index, edge_attr, W1, as1, ad1, We1, ae1, b1, g1, be1,
           W2, as2, ad2, We2, ae2, b2, g2, be2):
    src, dst = edge_index[0], edge_index[1]
    Ve1 = (We1.reshape(6, H1, C1) * ae1).sum(-1)
    Ve2 = (We2.reshape(6, 1, C2) * ae2).sum(-1)

    parts = _easum_parts(edge_index, edge_attr)
    ael1, ael2 = _aeloop_tc(parts, Ve1, Ve2)
    aee1, aee2 = _ae_edge_tc(edge_attr, Ve1, Ve2)

    xe1, adst1 = _prologue_tc(x, W1, _fold(as1), _fold(ad1))
    adst1_flat = jnp.concatenate(
        [adst1.reshape(-1), jnp.zeros(((_NN1 - N) * H1,), jnp.float32)])
    oe1 = _EP1(src, dst, aee1.reshape(-1), xe1, adst1_flat,
               jnp.zeros((_NN1, H1 * C1 + 16), jnp.float32))
    h1 = _epilogue_tc(oe1, xe1, adst1, ael1, b1, g1, be1, H1, C1, True)

    xe2, adst2 = _prologue_tc(h1, W2, _fold(as2), _fold(ad2))
    adst2_flat = jnp.concatenate(
        [adst2.reshape(-1), jnp.zeros((_NN2 - N,), jnp.float32)])
    oe2 = _EP2(src, dst, aee2.reshape(-1), xe2, adst2_flat,
               jnp.zeros((_NN2, C2 + 16), jnp.float32))
    return _epilogue_tc(oe2, xe2, adst2, ael2, b2, g2, be2, 1, C2, False)
